# baseline (device time: 449044 ns/iter reference)
import jax
import jax.numpy as jnp
from jax import lax
from jax.experimental import pallas as pl
from jax.experimental.pallas import tpu as pltpu

N_DEV = 16


def kernel(x, W1, W2):
    m, d = x.shape
    f = W1.shape[1]

    def body(x_ref, W1_ref, W2_ref, out_ref,
             xga_ref, p_ref, rsbuf_ref,
             ag_send, ag_recv, rs_send, rs_recv):
        my = lax.axis_index("i")
        right = (my + 1) % N_DEV
        left = (my + N_DEV - 1) % N_DEV

        barrier_sem = pltpu.get_barrier_semaphore()
        for nbr in (left, right):
            pl.semaphore_signal(
                barrier_sem, inc=1,
                device_id=(nbr,), device_id_type=pl.DeviceIdType.MESH,
            )
        pl.semaphore_wait(barrier_sem, 2)

        xga_ref[0] = x_ref[...]

        def compute_chunk(k):
            h = jnp.dot(xga_ref[k], W1_ref[...],
                        preferred_element_type=jnp.float32)
            h = h * jax.nn.sigmoid(h)
            p_ref[k] = jnp.dot(h, W2_ref[...],
                               preferred_element_type=jnp.float32)

        compute_chunk(0)

        for hp in range(N_DEV - 1):
            rdma = pltpu.make_async_remote_copy(
                src_ref=xga_ref.at[hp],
                dst_ref=xga_ref.at[hp + 1],
                send_sem=ag_send.at[hp],
                recv_sem=ag_recv.at[hp],
                device_id=(right,),
                device_id_type=pl.DeviceIdType.MESH,
            )
            rdma.start()
            rdma.wait()
            compute_chunk(hp + 1)

        for s in range(N_DEV - 1):
            rdma = pltpu.make_async_remote_copy(
                src_ref=p_ref.at[s + 1],
                dst_ref=rsbuf_ref.at[s],
                send_sem=rs_send.at[s],
                recv_sem=rs_recv.at[s],
                device_id=(right,),
                device_id_type=pl.DeviceIdType.MESH,
            )
            rdma.start()
            rdma.wait()
            if s < N_DEV - 2:
                p_ref[s + 2] = p_ref[s + 2] + rsbuf_ref[s]
            else:
                out_ref[...] = p_ref[0] + rsbuf_ref[s]

    return pl.pallas_call(
        body,
        out_shape=jax.ShapeDtypeStruct((m, d), jnp.float32),
        in_specs=[
            pl.BlockSpec(memory_space=pltpu.VMEM),
            pl.BlockSpec(memory_space=pltpu.VMEM),
            pl.BlockSpec(memory_space=pltpu.VMEM),
        ],
        out_specs=pl.BlockSpec(memory_space=pltpu.VMEM),
        scratch_shapes=[
            pltpu.VMEM((N_DEV, m, d), jnp.float32),
            pltpu.VMEM((N_DEV, m, d), jnp.float32),
            pltpu.VMEM((N_DEV - 1, m, d), jnp.float32),
            pltpu.SemaphoreType.DMA((N_DEV - 1,)),
            pltpu.SemaphoreType.DMA((N_DEV - 1,)),
            pltpu.SemaphoreType.DMA((N_DEV - 1,)),
            pltpu.SemaphoreType.DMA((N_DEV - 1,)),
        ],
        compiler_params=pltpu.CompilerParams(
            collective_id=0,
            vmem_limit_bytes=100 * 1024 * 1024,
        ),
    )(x, W1, W2)


# device time: 194343 ns/iter; 2.3106x vs baseline; 2.3106x over previous
import jax
import jax.numpy as jnp
from jax import lax
from jax.experimental import pallas as pl
from jax.experimental.pallas import tpu as pltpu

N_DEV = 16


def kernel(x, W1, W2):
    m, d = x.shape
    hm = m // 2

    def body(x_ref, W1_ref, W2_ref, out_ref,
             xgaR, xgaL, pR, pL,
             xsR, xrR, asR, arR,
             xsL, xrL, asL, arL):
        my = lax.axis_index("i")
        right = (my + 1) % N_DEV
        left = (my + N_DEV - 1) % N_DEV

        barrier_sem = pltpu.get_barrier_semaphore()
        for nbr in (left, right):
            pl.semaphore_signal(
                barrier_sem, inc=1,
                device_id=(nbr,), device_id_type=pl.DeviceIdType.MESH,
            )
        pl.semaphore_wait(barrier_sem, 2)

        def f(xc):
            h = jnp.dot(xc, W1_ref[...], preferred_element_type=jnp.float32)
            h = h * jax.nn.sigmoid(h)
            return jnp.dot(h, W2_ref[...], preferred_element_type=jnp.float32)

        def mk_x(buf, ss, rs, s, tgt):
            return pltpu.make_async_remote_copy(
                src_ref=buf.at[s], dst_ref=buf.at[s + 1],
                send_sem=ss.at[s], recv_sem=rs.at[s],
                device_id=(tgt,), device_id_type=pl.DeviceIdType.MESH,
            )

        def mk_acc(p, ss, rs, s, tgt):
            return pltpu.make_async_remote_copy(
                src_ref=p.at[s], dst_ref=p.at[s + 1],
                send_sem=ss.at[s], recv_sem=rs.at[s],
                device_id=(tgt,), device_id_type=pl.DeviceIdType.MESH,
            )

        xgaR[0] = x_ref[:hm, :]
        xgaL[0] = x_ref[hm:, :]
        pR[0] = f(xgaR[0])
        pL[0] = f(xgaL[0])

        xdR = [None] * (N_DEV - 1)
        xdL = [None] * (N_DEV - 1)
        adR = [None] * N_DEV
        adL = [None] * N_DEV

        xdR[0] = mk_x(xgaR, xsR, xrR, 0, right)
        xdR[0].start()
        xdL[0] = mk_x(xgaL, xsL, xrL, 0, left)
        xdL[0].start()
        adR[0] = mk_acc(pR, asR, arR, 0, right)
        adR[0].start()
        adL[0] = mk_acc(pL, asL, arL, 0, left)
        adL[0].start()

        for s in range(1, N_DEV):
            xdR[s - 1].wait_recv()
            if s < N_DEV - 1:
                xdR[s] = mk_x(xgaR, xsR, xrR, s, right)
                xdR[s].start()
            xdL[s - 1].wait_recv()
            if s < N_DEV - 1:
                xdL[s] = mk_x(xgaL, xsL, xrL, s, left)
                xdL[s].start()

            fR = f(xgaR[s])
            fL = f(xgaL[s])

            adR[s - 1].wait_recv()
            pR[s] = pR[s] + fR
            adR[s] = mk_acc(pR, asR, arR, s, right)
            adR[s].start()

            adL[s - 1].wait_recv()
            pL[s] = pL[s] + fL
            adL[s] = mk_acc(pL, asL, arL, s, left)
            adL[s].start()

        adR[N_DEV - 1].wait_recv()
        out_ref[:hm, :] = pR[N_DEV]
        adL[N_DEV - 1].wait_recv()
        out_ref[hm:, :] = pL[N_DEV]

        for d_ in xdR + xdL + adR + adL:
            d_.wait_send()

    return pl.pallas_call(
        body,
        out_shape=jax.ShapeDtypeStruct((m, d), jnp.float32),
        in_specs=[
            pl.BlockSpec(memory_space=pltpu.VMEM),
            pl.BlockSpec(memory_space=pltpu.VMEM),
            pl.BlockSpec(memory_space=pltpu.VMEM),
        ],
        out_specs=pl.BlockSpec(memory_space=pltpu.VMEM),
        scratch_shapes=[
            pltpu.VMEM((N_DEV, hm, d), jnp.float32),
            pltpu.VMEM((N_DEV, hm, d), jnp.float32),
            pltpu.VMEM((N_DEV + 1, hm, d), jnp.float32),
            pltpu.VMEM((N_DEV + 1, hm, d), jnp.float32),
            pltpu.SemaphoreType.DMA((N_DEV - 1,)),
            pltpu.SemaphoreType.DMA((N_DEV - 1,)),
            pltpu.SemaphoreType.DMA((N_DEV,)),
            pltpu.SemaphoreType.DMA((N_DEV,)),
            pltpu.SemaphoreType.DMA((N_DEV - 1,)),
            pltpu.SemaphoreType.DMA((N_DEV - 1,)),
            pltpu.SemaphoreType.DMA((N_DEV,)),
            pltpu.SemaphoreType.DMA((N_DEV,)),
        ],
        compiler_params=pltpu.CompilerParams(
            collective_id=0,
            vmem_limit_bytes=110 * 1024 * 1024,
        ),
    )(x, W1, W2)


# device time: 190140 ns/iter; 2.3616x vs baseline; 1.0221x over previous
import jax
import jax.numpy as jnp
from jax import lax
from jax.experimental import pallas as pl
from jax.experimental.pallas import tpu as pltpu

N_DEV = 16
SUBK = 2


def kernel(x, W1, W2):
    m, d = x.shape
    n_pipes = 2 * SUBK
    sm = m // n_pipes

    def body(x_ref, W1_ref, W2_ref, out_ref,
             xga, p, xs, xr, acs, acr):
        my = lax.axis_index("i")
        right = (my + 1) % N_DEV
        left = (my + N_DEV - 1) % N_DEV

        barrier_sem = pltpu.get_barrier_semaphore()
        for nbr in (left, right):
            pl.semaphore_signal(
                barrier_sem, inc=1,
                device_id=(nbr,), device_id_type=pl.DeviceIdType.MESH,
            )
        pl.semaphore_wait(barrier_sem, 2)

        tgts = [right, left] * SUBK

        def rowoff(i):
            return (i % 2) * (m // 2) + (i // 2) * sm

        def f(xc):
            h = jnp.dot(xc, W1_ref[...], preferred_element_type=jnp.float32)
            h = h * jax.nn.sigmoid(h)
            return jnp.dot(h, W2_ref[...], preferred_element_type=jnp.float32)

        def mk_x(i, s):
            return pltpu.make_async_remote_copy(
                src_ref=xga.at[i, s], dst_ref=xga.at[i, s + 1],
                send_sem=xs.at[i, s], recv_sem=xr.at[i, s],
                device_id=(tgts[i],), device_id_type=pl.DeviceIdType.MESH,
            )

        def mk_acc(i, s):
            return pltpu.make_async_remote_copy(
                src_ref=p.at[i, s], dst_ref=p.at[i, s + 1],
                send_sem=acs.at[i, s], recv_sem=acr.at[i, s],
                device_id=(tgts[i],), device_id_type=pl.DeviceIdType.MESH,
            )

        xd = [[None] * (N_DEV - 1) for _ in range(n_pipes)]
        ad = [[None] * N_DEV for _ in range(n_pipes)]

        for i in range(n_pipes):
            off = rowoff(i)
            xga[i, 0] = x_ref[off:off + sm, :]
            xd[i][0] = mk_x(i, 0)
            xd[i][0].start()
        for i in range(n_pipes):
            p[i, 0] = f(xga[i, 0])
            ad[i][0] = mk_acc(i, 0)
            ad[i][0].start()

        for s in range(1, N_DEV):
            for i in range(n_pipes):
                xd[i][s - 1].wait_recv()
                if s < N_DEV - 1:
                    xd[i][s] = mk_x(i, s)
                    xd[i][s].start()
                fi = f(xga[i, s])
                ad[i][s - 1].wait_recv()
                p[i, s] = p[i, s] + fi
                ad[i][s] = mk_acc(i, s)
                ad[i][s].start()

        for i in range(n_pipes):
            ad[i][N_DEV - 1].wait_recv()
            off = rowoff(i)
            out_ref[off:off + sm, :] = p[i, N_DEV]

        for descs in xd + ad:
            for d_ in descs:
                d_.wait_send()

    return pl.pallas_call(
        body,
        out_shape=jax.ShapeDtypeStruct((m, d), jnp.float32),
        in_specs=[
            pl.BlockSpec(memory_space=pltpu.VMEM),
            pl.BlockSpec(memory_space=pltpu.VMEM),
            pl.BlockSpec(memory_space=pltpu.VMEM),
        ],
        out_specs=pl.BlockSpec(memory_space=pltpu.VMEM),
        scratch_shapes=[
            pltpu.VMEM((n_pipes, N_DEV, sm, d), jnp.float32),
            pltpu.VMEM((n_pipes, N_DEV + 1, sm, d), jnp.float32),
            pltpu.SemaphoreType.DMA((n_pipes, N_DEV - 1)),
            pltpu.SemaphoreType.DMA((n_pipes, N_DEV - 1)),
            pltpu.SemaphoreType.DMA((n_pipes, N_DEV)),
            pltpu.SemaphoreType.DMA((n_pipes, N_DEV)),
        ],
        compiler_params=pltpu.CompilerParams(
            collective_id=0,
            vmem_limit_bytes=110 * 1024 * 1024,
        ),
    )(x, W1, W2)


# device time: 184565 ns/iter; 2.4330x vs baseline; 1.0302x over previous
import jax
import jax.numpy as jnp
from jax import lax
from jax.experimental import pallas as pl
from jax.experimental.pallas import tpu as pltpu

N_DEV = 16
SUBK = 2


def kernel(x, W1, W2):
    m, d = x.shape
    n_pipes = 2 * SUBK
    sm = m // n_pipes

    def body(x_ref, W1_ref, W2_ref, out_ref,
             xga, p, xs, xr, acs, acr):
        my = lax.axis_index("i")
        right = (my + 1) % N_DEV
        left = (my + N_DEV - 1) % N_DEV

        barrier_sem = pltpu.get_barrier_semaphore()
        for nbr in (left, right):
            pl.semaphore_signal(
                barrier_sem, inc=1,
                device_id=(nbr,), device_id_type=pl.DeviceIdType.MESH,
            )
        pl.semaphore_wait(barrier_sem, 2)

        tgts = [right, left] * SUBK

        def rowoff(i):
            return (i % 2) * (m // 2) + (i // 2) * sm

        def f(xc):
            h = jnp.dot(xc, W1_ref[...], preferred_element_type=jnp.float32)
            h = h * jax.nn.sigmoid(h)
            return jnp.dot(h, W2_ref[...], preferred_element_type=jnp.float32)

        def mk_x(i, s):
            return pltpu.make_async_remote_copy(
                src_ref=xga.at[i, s], dst_ref=xga.at[i, s + 1],
                send_sem=xs.at[i, s], recv_sem=xr.at[i, s],
                device_id=(tgts[i],), device_id_type=pl.DeviceIdType.MESH,
            )

        def mk_acc(i, s):
            return pltpu.make_async_remote_copy(
                src_ref=p.at[i, s], dst_ref=p.at[i, s + 1],
                send_sem=acs.at[i, s], recv_sem=acr.at[i, s],
                device_id=(tgts[i],), device_id_type=pl.DeviceIdType.MESH,
            )

        xd = [[None] * (N_DEV - 1) for _ in range(n_pipes)]
        ad = [[None] * N_DEV for _ in range(n_pipes)]

        for i in range(n_pipes):
            off = rowoff(i)
            xga[i, 0] = x_ref[off:off + sm, :]
            xd[i][0] = mk_x(i, 0)
            xd[i][0].start()
        for i in range(n_pipes):
            p[i, 0] = f(xga[i, 0])

        for s in range(1, N_DEV):
            for i in range(n_pipes):
                xd[i][s - 1].wait_recv()
                if s < N_DEV - 1:
                    xd[i][s] = mk_x(i, s)
                    xd[i][s].start()
                fi = f(xga[i, s])
                if s >= 2:
                    ad[i][s - 1].wait_recv()
                    p[i, s] = p[i, s] + fi
                else:
                    p[i, s] = fi
                ad[i][s] = mk_acc(i, s)
                ad[i][s].start()

        for i in range(n_pipes):
            ad[i][N_DEV - 1].wait_recv()
            off = rowoff(i)
            out_ref[off:off + sm, :] = p[i, N_DEV] + p[i, 0]

        for descs in xd + ad:
            for d_ in descs:
                if d_ is not None:
                    d_.wait_send()

    return pl.pallas_call(
        body,
        out_shape=jax.ShapeDtypeStruct((m, d), jnp.float32),
        in_specs=[
            pl.BlockSpec(memory_space=pltpu.VMEM),
            pl.BlockSpec(memory_space=pltpu.VMEM),
            pl.BlockSpec(memory_space=pltpu.VMEM),
        ],
        out_specs=pl.BlockSpec(memory_space=pltpu.VMEM),
        scratch_shapes=[
            pltpu.VMEM((n_pipes, N_DEV, sm, d), jnp.float32),
            pltpu.VMEM((n_pipes, N_DEV + 1, sm, d), jnp.float32),
            pltpu.SemaphoreType.DMA((n_pipes, N_DEV - 1)),
            pltpu.SemaphoreType.DMA((n_pipes, N_DEV - 1)),
            pltpu.SemaphoreType.DMA((n_pipes, N_DEV)),
            pltpu.SemaphoreType.DMA((n_pipes, N_DEV)),
        ],
        compiler_params=pltpu.CompilerParams(
            collective_id=0,
            vmem_limit_bytes=110 * 1024 * 1024,
        ),
    )(x, W1, W2)
